# baseline (speedup 1.0000x reference)
import functools

import jax
import jax.numpy as jnp
from jax import lax
from jax.experimental import pallas as pl
from jax.experimental.pallas import tpu as pltpu
from jax.experimental.pallas import tpu_sc as plsc

NN = 10000
NPAD = 10240
EE = 320000
FM = 128
NL = 4
NC = 2
NS = 16
NW = NC * NS
EPW = EE // NW
CH = 128
EPP = 10112
NCHUNK = EPP // CH
PADR = 10100
PADC = 10000
RPT = NPAD // NS

_mesh = plsc.VectorSubcoreMesh(core_axis_name="c", subcore_axis_name="s")


@functools.partial(
    pl.kernel,
    out_type=(jax.ShapeDtypeStruct((NW * EPP,), jnp.float32),
              jax.ShapeDtypeStruct((NC * NPAD,), jnp.float32)),
    mesh=_mesh,
    scratch_types=[
        pltpu.VMEM((2, CH), jnp.int32),
        pltpu.VMEM((2, CH), jnp.int32),
        pltpu.VMEM((CH,), jnp.int32),
        pltpu.VMEM((CH,), jnp.int32),
        pltpu.VMEM((CH,), jnp.int32),
        pltpu.VMEM((CH,), jnp.int32),
        pltpu.VMEM((CH,), jnp.float32),
        pltpu.VMEM((CH,), jnp.float32),
        pltpu.VMEM_SHARED((NPAD,), jnp.float32),
        pltpu.SemaphoreType.DMA,
        pltpu.SemaphoreType.DMA,
        pltpu.SemaphoreType.DMA,
        pltpu.SemaphoreType.DMA,
        pltpu.SemaphoreType.DMA,
        pltpu.SemaphoreType.DMA,
        pltpu.SemaphoreType.DMA,
        pltpu.SemaphoreType.DMA,
    ],
)
def _sc_edge_vals(mat_hbm, rc_hbm, zrow_hbm,
                  ev_hbm, degp_hbm,
                  i0, i1, fi0, fi1, sr0, sr1, e0, e1, acc,
                  ii0, ii1, g0, g1, o0, o1, d0, d1):
    cid = lax.axis_index("c")
    sid = lax.axis_index("s")
    wid = sid * NC + cid
    pltpu.sync_copy(zrow_hbm, acc.at[pl.ds(sid * RPT, RPT)])
    plsc.subcore_barrier()

    ib = (i0, i1)
    fib = (fi0, fi1)
    srb = (sr0, sr1)
    eb = (e0, e1)
    gsem = (g0, g1)
    osem = (o0, o1)
    dsem = (d0, d1)

    def fetch_idx(j, b, sem):
        pltpu.async_copy(rc_hbm.at[wid, j], ib[b], sem)

    def wait_idx(b, sem):
        pltpu.make_async_copy(rc_hbm.at[0, 0], ib[b], sem).wait()

    def front(b):
        for v in range(CH // 16):
            sl = pl.ds(v * 16, 16)
            r = ib[b][0, sl]
            fib[b][sl] = jnp.minimum(r * NN + ib[b][1, sl], NN * NN - 1)
            srb[b][sl] = r
        pltpu.async_copy(mat_hbm.at[fib[b]], eb[b], gsem[b])

    def back(j, b):
        pltpu.make_async_copy(mat_hbm.at[fi0], eb[b], gsem[b]).wait()
        base = wid * EPP + j * CH
        pltpu.async_copy(eb[b], ev_hbm.at[pl.ds(base, CH)], osem[b])
        pltpu.async_copy(eb[b], acc.at[srb[b]], dsem[b], add=True)

    def wait_back(b):
        pltpu.make_async_copy(eb[b], ev_hbm.at[pl.ds(0, CH)], osem[b]).wait()
        pltpu.make_async_copy(eb[b], acc.at[sr0], dsem[b]).wait()

    pltpu.sync_copy(rc_hbm.at[wid, 0], i0)
    front(0)
    fetch_idx(1, 1, ii1)

    def pair(t, carry):
        a = 2 * t
        wait_idx(1, ii1)
        front(1)
        fetch_idx(a + 2, 0, ii0)
        back(a, 0)
        wait_idx(0, ii0)
        wait_back(0)
        front(0)
        fetch_idx(a + 3, 1, ii1)
        back(a + 1, 1)
        wait_back(1)
        return carry

    lax.fori_loop(0, (NCHUNK - 1) // 2, pair, 0)
    back(NCHUNK - 1, 0)
    wait_back(0)
    wait_idx(1, ii1)
    plsc.subcore_barrier()
    pltpu.sync_copy(acc.at[pl.ds(sid * RPT, RPT)],
                    degp_hbm.at[pl.ds(cid * NPAD + sid * RPT, RPT)])


@functools.partial(
    pl.kernel,
    out_type=jax.ShapeDtypeStruct((NW * EPP,), jnp.float32),
    mesh=_mesh,
    scratch_types=[
        pltpu.VMEM((2, CH), jnp.int32),
        pltpu.VMEM((2, CH), jnp.int32),
        pltpu.VMEM((CH,), jnp.float32),
        pltpu.VMEM((CH,), jnp.float32),
        pltpu.VMEM((CH,), jnp.float32),
        pltpu.VMEM((CH,), jnp.float32),
        pltpu.VMEM((CH,), jnp.float32),
        pltpu.VMEM((CH,), jnp.float32),
        pltpu.VMEM((CH,), jnp.float32),
        pltpu.VMEM((CH,), jnp.float32),
        pltpu.SemaphoreType.DMA,
        pltpu.SemaphoreType.DMA,
        pltpu.SemaphoreType.DMA,
        pltpu.SemaphoreType.DMA,
        pltpu.SemaphoreType.DMA,
        pltpu.SemaphoreType.DMA,
    ],
)
def _sc_edge_norm(dis_hbm, rc_hbm, ev_hbm,
                  nv_hbm,
                  i0, i1, e0, e1, dr0, dr1, dc0, dc1, n0, n1,
                  ii0, ii1, g0, g1, o0, o1):
    cid = lax.axis_index("c")
    sid = lax.axis_index("s")
    wid = sid * NC + cid

    ib = (i0, i1)
    eb = (e0, e1)
    drb = (dr0, dr1)
    dcb = (dc0, dc1)
    nb = (n0, n1)
    isem = (ii0, ii1)
    gsem = (g0, g1)
    osem = (o0, o1)

    def fetch_idx(j, b):
        base = wid * EPP + j * CH
        pltpu.async_copy(rc_hbm.at[wid, j], ib[b], isem[b])
        pltpu.async_copy(ev_hbm.at[pl.ds(base, CH)], eb[b], isem[b])

    def wait_idx(b):
        pltpu.make_async_copy(rc_hbm.at[0, 0], ib[b], isem[b]).wait()
        pltpu.make_async_copy(ev_hbm.at[pl.ds(0, CH)], eb[b], isem[b]).wait()

    def front(b):
        pltpu.async_copy(dis_hbm.at[ib[b].at[0]], drb[b], gsem[b])
        pltpu.async_copy(dis_hbm.at[ib[b].at[1]], dcb[b], gsem[b])

    def back(j, b):
        pltpu.make_async_copy(dis_hbm.at[i0.at[0]], drb[b], gsem[b]).wait()
        pltpu.make_async_copy(dis_hbm.at[i0.at[0]], dcb[b], gsem[b]).wait()
        for v in range(CH // 16):
            sl = pl.ds(v * 16, 16)
            nb[b][sl] = drb[b][sl] * eb[b][sl] * dcb[b][sl]
        base = wid * EPP + j * CH
        pltpu.async_copy(nb[b], nv_hbm.at[pl.ds(base, CH)], osem[b])

    def wait_back(b):
        pltpu.make_async_copy(nb[b], nv_hbm.at[pl.ds(0, CH)], osem[b]).wait()

    fetch_idx(0, 0)
    wait_idx(0)
    front(0)
    fetch_idx(1, 1)

    def pair(t, carry):
        a = 2 * t
        wait_idx(1)
        front(1)
        back(a, 0)
        fetch_idx(a + 2, 0)
        wait_idx(0)
        front(0)
        wait_back(0)
        back(a + 1, 1)
        fetch_idx(a + 3, 1)
        wait_back(1)
        return carry

    lax.fori_loop(0, (NCHUNK - 1) // 2, pair, 0)
    back(NCHUNK - 1, 0)
    wait_back(0)
    wait_idx(1)


@functools.partial(
    pl.kernel,
    out_type=jax.ShapeDtypeStruct((NC * NPAD, FM), jnp.float32),
    mesh=_mesh,
    scratch_types=[
        pltpu.VMEM((2, CH), jnp.int32),
        pltpu.VMEM((2, CH), jnp.int32),
        pltpu.VMEM((CH,), jnp.float32),
        pltpu.VMEM((CH,), jnp.float32),
        pltpu.VMEM((CH, FM), jnp.float32),
        pltpu.VMEM((CH, FM), jnp.float32),
        pltpu.VMEM((CH,), jnp.int32),
        pltpu.VMEM((CH,), jnp.int32),
        pltpu.VMEM_SHARED((NPAD, FM), jnp.float32),
        pltpu.SemaphoreType.DMA,
        pltpu.SemaphoreType.DMA,
        pltpu.SemaphoreType.DMA,
        pltpu.SemaphoreType.DMA,
        pltpu.SemaphoreType.DMA,
        pltpu.SemaphoreType.DMA,
    ],
)
def _sc_spmm(h_hbm, rc_hbm, nvh_hbm, zblk_hbm,
             out_hbm,
             i0, i1, n0, n1, buf0, buf1, si0, si1, acc,
             g0, g1, s0, s1, ii0, ii1):
    cid = lax.axis_index("c")
    sid = lax.axis_index("s")
    wid = sid * NC + cid
    pltpu.sync_copy(zblk_hbm, acc.at[pl.ds(sid * RPT, RPT)])
    plsc.subcore_barrier()

    def scale(buf, nbuf):
        def edge16(g, ecarry):
            vg = nbuf[pl.ds(g * 16, 16)]
            for jj in range(16):
                s = jnp.full((16,), vg[jj], jnp.float32)
                k = g * 16 + jj
                for v in range(FM // 16):
                    sl = pl.ds(v * 16, 16)
                    buf[k, sl] = buf[k, sl] * s
            return ecarry

        lax.fori_loop(0, CH // 16, edge16, 0)

    def fetch_idx(j, ibuf, nbuf, sem):
        pltpu.async_copy(rc_hbm.at[wid, j], ibuf, sem)
        pltpu.async_copy(nvh_hbm.at[wid, j], nbuf, sem)

    def wait_idx(ibuf, nbuf, sem):
        pltpu.make_async_copy(rc_hbm.at[0, 0], ibuf, sem).wait()
        pltpu.make_async_copy(nvh_hbm.at[0, 0], nbuf, sem).wait()

    def gather(ibuf, buf, sem):
        pltpu.async_copy(h_hbm.at[ibuf.at[1]], buf, sem)

    def wait_gather(buf, sem):
        pltpu.make_async_copy(h_hbm.at[i0.at[1]], buf, sem).wait()

    def sicopy(ibuf, sbuf):
        for v in range(CH // 16):
            sl = pl.ds(v * 16, 16)
            sbuf[sl] = ibuf[0, sl]

    def scatter(sbuf, buf, sem):
        pltpu.async_copy(buf, acc.at[sbuf], sem, add=True)

    def wait_scatter(buf, sem):
        pltpu.make_async_copy(buf, acc.at[si0], sem).wait()

    pltpu.sync_copy(rc_hbm.at[wid, 0], i0)
    pltpu.sync_copy(nvh_hbm.at[wid, 0], n0)
    gather(i0, buf0, g0)
    fetch_idx(1, i1, n1, ii1)

    def pair(t, carry):
        a = 2 * t
        wait_idx(i1, n1, ii1)
        gather(i1, buf1, g1)
        wait_gather(buf0, g0)
        scale(buf0, n0)
        sicopy(i0, si0)
        scatter(si0, buf0, s0)
        fetch_idx(a + 2, i0, n0, ii0)
        wait_gather(buf1, g1)
        scale(buf1, n1)
        sicopy(i1, si1)
        scatter(si1, buf1, s1)
        fetch_idx(a + 3, i1, n1, ii1)
        wait_scatter(buf0, s0)
        wait_idx(i0, n0, ii0)
        gather(i0, buf0, g0)
        wait_scatter(buf1, s1)
        return carry

    lax.fori_loop(0, (NCHUNK - 1) // 2, pair, 0)
    wait_gather(buf0, g0)
    scale(buf0, n0)
    sicopy(i0, si0)
    scatter(si0, buf0, s0)
    wait_scatter(buf0, s0)
    wait_idx(i1, n1, ii1)
    plsc.subcore_barrier()
    pltpu.sync_copy(acc.at[pl.ds(sid * RPT, RPT)],
                    out_hbm.at[pl.ds(cid * NPAD + sid * RPT, RPT)])


def _tc_dis_body(dp_ref, dis_ref, sl_ref):
    deg = dp_ref[0] + dp_ref[1] + (1.0 + 1e-6)
    dis = jnp.minimum(lax.rsqrt(deg), 1000.0)
    dis_ref[...] = dis
    sl_ref[...] = dis * dis


def _tc_dis(degp):
    return pl.pallas_call(
        _tc_dis_body,
        out_shape=[jax.ShapeDtypeStruct((8, NPAD // 8), jnp.float32),
                   jax.ShapeDtypeStruct((8, NPAD // 8), jnp.float32)],
    )(degp)


BR = 1280
GRID = NPAD // BR


def _tc_layer_body(p_ref, h_ref, sl_ref, wt_ref, b_ref, hout_ref, cs_ref):
    gid = pl.program_id(0)
    support = p_ref[0] + p_ref[1] + sl_ref[...] * h_ref[...]
    hh = jnp.dot(support, wt_ref[...], preferred_element_type=jnp.float32)
    hh = jnp.maximum(hh + b_ref[...], 0.0)
    rowid = gid * BR + lax.broadcasted_iota(jnp.int32, (BR, 1), 0)
    hh = jnp.where(rowid < NN, hh, 0.0)
    hout_ref[...] = hh

    @pl.when(gid == 0)
    def _():
        cs_ref[...] = jnp.zeros_like(cs_ref)

    cs_ref[...] += jnp.sum(hh, axis=0, keepdims=True)


def _tc_layer(partials, h, slv, wt, b):
    return pl.pallas_call(
        _tc_layer_body,
        grid=(GRID,),
        in_specs=[
            pl.BlockSpec((NC, BR, FM), lambda i: (0, i, 0)),
            pl.BlockSpec((BR, FM), lambda i: (i, 0)),
            pl.BlockSpec((BR, 1), lambda i: (i, 0)),
            pl.BlockSpec((FM, FM), lambda i: (0, 0)),
            pl.BlockSpec((1, FM), lambda i: (0, 0)),
        ],
        out_specs=[
            pl.BlockSpec((BR, FM), lambda i: (i, 0)),
            pl.BlockSpec((1, FM), lambda i: (0, 0)),
        ],
        out_shape=[jax.ShapeDtypeStruct((NPAD, FM), jnp.float32),
                   jax.ShapeDtypeStruct((1, FM), jnp.float32)],
    )(partials, h, slv, wt, b)


def _tc_attn_body(cs_ref, f1t_ref, f1b_ref, f2_ref, f2b_ref, cw_ref,
                  coef_ref):
    inv = 1.0 / (NN * FM)
    a1 = f1b_ref[...]
    for j in range(NL):
        ca_j = jnp.sum(cs_ref[j, :]) * inv
        a1 = a1 + ca_j * f1t_ref[j:j + 1, :]
    a1 = jnp.maximum(a1, 0.0)
    lane = lax.broadcasted_iota(jnp.int32, (1, FM), 1)
    out = jnp.zeros((1, FM), jnp.float32)
    for j in range(NL):
        oh = (lane == j).astype(jnp.float32)
        zj = jnp.sum(f2_ref[j:j + 1, :] * a1) + jnp.sum(f2b_ref[...] * oh)
        cj = jax.nn.sigmoid(zj) * jnp.sum(cw_ref[...] * oh)
        out = out + cj * oh
    coef_ref[...] = out


def _tc_attn(colsums, f1t, f1b, f2p, f2b, cwp):
    return pl.pallas_call(
        _tc_attn_body,
        out_shape=jax.ShapeDtypeStruct((1, FM), jnp.float32),
    )(colsums, f1t, f1b, f2p, f2b, cwp)


def _tc_combine_body(h0, h1, h2, h3, coef_ref, cb_ref, out_ref):
    lane = lax.broadcasted_iota(jnp.int32, (1, FM), 1)
    c = coef_ref[...]
    oh0 = (lane == 0).astype(jnp.float32)
    acc = jnp.sum(cb_ref[...] * oh0) + jnp.zeros_like(h0[...])
    hs = (h0, h1, h2, h3)
    for j in range(NL):
        cj = jnp.sum(c * (lane == j).astype(jnp.float32))
        acc = acc + cj * hs[j][...]
    out_ref[...] = acc


def _tc_combine(h0, h1, h2, h3, coef, cbp):
    blk = pl.BlockSpec((BR, FM), lambda i: (i, 0))
    one = pl.BlockSpec((1, FM), lambda i: (0, 0))
    return pl.pallas_call(
        _tc_combine_body,
        grid=(GRID,),
        in_specs=[blk, blk, blk, blk, one, one],
        out_specs=blk,
        out_shape=jax.ShapeDtypeStruct((NPAD, FM), jnp.float32),
    )(h0, h1, h2, h3, coef, cbp)


def kernel(x, edge_index, matrix, Ws, bs, fc1_w, fc1_b, fc2_w, fc2_b,
           cnn_w, cnn_b):
    rows = edge_index[0]
    cols = edge_index[1]
    matflat = matrix.reshape(-1)
    zrow = jnp.zeros((RPT,), jnp.float32)
    zblk = jnp.zeros((RPT, FM), jnp.float32)
    rpad = jnp.full((NW, EPP - EPW), PADR, jnp.int32)
    cpad = jnp.full((NW, EPP - EPW), PADC, jnp.int32)
    rows2 = jnp.concatenate([rows.reshape(NW, EPW), rpad], axis=1)
    cols2 = jnp.concatenate([cols.reshape(NW, EPW), cpad], axis=1)
    rc = jnp.stack([rows2.reshape(NW, NCHUNK, CH),
                    cols2.reshape(NW, NCHUNK, CH)], axis=2)
    rc = jnp.pad(rc, ((0, 0), (0, 1), (0, 0), (0, 0)))

    ev, degp = _sc_edge_vals(matflat, rc, zrow)
    dis8, sl8 = _tc_dis(degp.reshape(NC, 8, NPAD // 8))
    dis = dis8.reshape(NPAD)
    slv = sl8.reshape(NPAD, 1)
    nv = _sc_edge_norm(dis, rc, ev)

    h = jnp.pad(x, ((0, NPAD - NN), (0, 0)))
    nvh = jnp.pad(nv.reshape(NW, NCHUNK, CH), ((0, 0), (0, 1), (0, 0)))
    hs, css = [], []
    for l in range(NL):
        partials = _sc_spmm(h, rc, nvh, zblk).reshape(NC, NPAD, FM)
        h, cs = _tc_layer(partials, h, slv, Ws[l].T, bs[l].reshape(1, FM))
        hs.append(h)
        css.append(cs)

    colsums = jnp.concatenate(css, axis=0)
    k1 = fc1_w.shape[0]
    f1t = jnp.pad(fc1_w.T, ((0, 0), (0, FM - k1)))
    f1b = jnp.pad(fc1_b, (0, FM - k1)).reshape(1, FM)
    f2p = jnp.pad(fc2_w, ((0, 0), (0, FM - k1)))
    f2b = jnp.pad(fc2_b, (0, FM - NL)).reshape(1, FM)
    cwp = jnp.pad(cnn_w, (0, FM - NL)).reshape(1, FM)
    cbp = jnp.pad(cnn_b, (0, FM - 1)).reshape(1, FM)

    coef = _tc_attn(colsums, f1t, f1b, f2p, f2b, cwp)
    outp = _tc_combine(hs[0], hs[1], hs[2], hs[3], coef, cbp)
    return outp[:NN]


# speedup vs baseline: 1.3694x; 1.3694x over previous
import functools

import jax
import jax.numpy as jnp
from jax import lax
from jax.experimental import pallas as pl
from jax.experimental.pallas import tpu as pltpu
from jax.experimental.pallas import tpu_sc as plsc

NN = 10000
NPAD = 10240
EE = 320000
FM = 128
NL = 4
NC = 2
NS = 16
NW = NC * NS
EPW = EE // NW
CH = 80
NCHUNK = EPW // CH
RPT = NPAD // NS

_mesh = plsc.VectorSubcoreMesh(core_axis_name="c", subcore_axis_name="s")


@functools.partial(
    pl.kernel,
    out_type=(jax.ShapeDtypeStruct((EE,), jnp.float32),
              jax.ShapeDtypeStruct((NC * NPAD,), jnp.float32)),
    mesh=_mesh,
    scratch_types=[
        pltpu.VMEM((2, CH), jnp.int32),
        pltpu.VMEM((2, CH), jnp.int32),
        pltpu.VMEM((CH,), jnp.int32),
        pltpu.VMEM((CH,), jnp.int32),
        pltpu.VMEM((CH,), jnp.int32),
        pltpu.VMEM((CH,), jnp.int32),
        pltpu.VMEM((CH,), jnp.float32),
        pltpu.VMEM((CH,), jnp.float32),
        pltpu.VMEM_SHARED((NPAD,), jnp.float32),
        pltpu.SemaphoreType.DMA,
        pltpu.SemaphoreType.DMA,
        pltpu.SemaphoreType.DMA,
        pltpu.SemaphoreType.DMA,
        pltpu.SemaphoreType.DMA,
        pltpu.SemaphoreType.DMA,
        pltpu.SemaphoreType.DMA,
        pltpu.SemaphoreType.DMA,
    ],
)
def _sc_edge_vals(mat_hbm, rc_hbm, zrow_hbm,
                  ev_hbm, degp_hbm,
                  i0, i1, fi0, fi1, sr0, sr1, e0, e1, acc,
                  ii0, ii1, g0, g1, o0, o1, d0, d1):
    cid = lax.axis_index("c")
    sid = lax.axis_index("s")
    wid = sid * NC + cid
    pltpu.sync_copy(zrow_hbm, acc.at[pl.ds(sid * RPT, RPT)])
    plsc.subcore_barrier()

    ib = (i0, i1)
    fib = (fi0, fi1)
    srb = (sr0, sr1)
    eb = (e0, e1)
    gsem = (g0, g1)
    osem = (o0, o1)
    dsem = (d0, d1)

    def fetch_idx(j, b, sem):
        pltpu.async_copy(rc_hbm.at[wid, j], ib[b], sem)

    def wait_idx(b, sem):
        pltpu.make_async_copy(rc_hbm.at[0, 0], ib[b], sem).wait()

    def front(b):
        for v in range(CH // 16):
            sl = pl.ds(v * 16, 16)
            r = ib[b][0, sl]
            fib[b][sl] = r * NN + ib[b][1, sl]
            srb[b][sl] = r
        pltpu.async_copy(mat_hbm.at[fib[b]], eb[b], gsem[b])

    def back(j, b):
        pltpu.make_async_copy(mat_hbm.at[fi0], eb[b], gsem[b]).wait()
        base = wid * EPW + j * CH
        pltpu.async_copy(eb[b], ev_hbm.at[pl.ds(base, CH)], osem[b])
        pltpu.async_copy(eb[b], acc.at[srb[b]], dsem[b], add=True)

    def wait_back(b):
        pltpu.make_async_copy(eb[b], ev_hbm.at[pl.ds(0, CH)], osem[b]).wait()
        pltpu.make_async_copy(eb[b], acc.at[sr0], dsem[b]).wait()

    pltpu.sync_copy(rc_hbm.at[wid, 0], i0)
    front(0)
    fetch_idx(1, 1, ii1)

    def pair(t, carry):
        a = 2 * t
        wait_idx(1, ii1)
        front(1)
        fetch_idx(a + 2, 0, ii0)
        back(a, 0)
        wait_idx(0, ii0)
        wait_back(0)
        front(0)
        fetch_idx(a + 3, 1, ii1)
        back(a + 1, 1)
        wait_back(1)
        return carry

    lax.fori_loop(0, (NCHUNK - 1) // 2, pair, 0)
    back(NCHUNK - 1, 0)
    wait_back(0)
    wait_idx(1, ii1)
    plsc.subcore_barrier()
    pltpu.sync_copy(acc.at[pl.ds(sid * RPT, RPT)],
                    degp_hbm.at[pl.ds(cid * NPAD + sid * RPT, RPT)])


@functools.partial(
    pl.kernel,
    out_type=jax.ShapeDtypeStruct((EE,), jnp.float32),
    mesh=_mesh,
    scratch_types=[
        pltpu.VMEM((2, CH), jnp.int32),
        pltpu.VMEM((2, CH), jnp.int32),
        pltpu.VMEM((CH,), jnp.float32),
        pltpu.VMEM((CH,), jnp.float32),
        pltpu.VMEM((CH,), jnp.float32),
        pltpu.VMEM((CH,), jnp.float32),
        pltpu.VMEM((CH,), jnp.float32),
        pltpu.VMEM((CH,), jnp.float32),
        pltpu.VMEM((CH,), jnp.float32),
        pltpu.VMEM((CH,), jnp.float32),
        pltpu.SemaphoreType.DMA,
        pltpu.SemaphoreType.DMA,
        pltpu.SemaphoreType.DMA,
        pltpu.SemaphoreType.DMA,
        pltpu.SemaphoreType.DMA,
        pltpu.SemaphoreType.DMA,
    ],
)
def _sc_edge_norm(dis_hbm, rc_hbm, ev_hbm,
                  nv_hbm,
                  i0, i1, e0, e1, dr0, dr1, dc0, dc1, n0, n1,
                  ii0, ii1, g0, g1, o0, o1):
    cid = lax.axis_index("c")
    sid = lax.axis_index("s")
    wid = sid * NC + cid

    ib = (i0, i1)
    eb = (e0, e1)
    drb = (dr0, dr1)
    dcb = (dc0, dc1)
    nb = (n0, n1)
    isem = (ii0, ii1)
    gsem = (g0, g1)
    osem = (o0, o1)

    def fetch_idx(j, b):
        base = wid * EPW + j * CH
        pltpu.async_copy(rc_hbm.at[wid, j], ib[b], isem[b])
        pltpu.async_copy(ev_hbm.at[pl.ds(base, CH)], eb[b], isem[b])

    def wait_idx(b):
        pltpu.make_async_copy(rc_hbm.at[0, 0], ib[b], isem[b]).wait()
        pltpu.make_async_copy(ev_hbm.at[pl.ds(0, CH)], eb[b], isem[b]).wait()

    def front(b):
        pltpu.async_copy(dis_hbm.at[ib[b].at[0]], drb[b], gsem[b])
        pltpu.async_copy(dis_hbm.at[ib[b].at[1]], dcb[b], gsem[b])

    def back(j, b):
        pltpu.make_async_copy(dis_hbm.at[i0.at[0]], drb[b], gsem[b]).wait()
        pltpu.make_async_copy(dis_hbm.at[i0.at[0]], dcb[b], gsem[b]).wait()
        for v in range(CH // 16):
            sl = pl.ds(v * 16, 16)
            nb[b][sl] = drb[b][sl] * eb[b][sl] * dcb[b][sl]
        base = wid * EPW + j * CH
        pltpu.async_copy(nb[b], nv_hbm.at[pl.ds(base, CH)], osem[b])

    def wait_back(b):
        pltpu.make_async_copy(nb[b], nv_hbm.at[pl.ds(0, CH)], osem[b]).wait()

    fetch_idx(0, 0)
    wait_idx(0)
    front(0)
    fetch_idx(1, 1)

    def pair(t, carry):
        a = 2 * t
        wait_idx(1)
        front(1)
        back(a, 0)
        fetch_idx(a + 2, 0)
        wait_idx(0)
        front(0)
        wait_back(0)
        back(a + 1, 1)
        fetch_idx(a + 3, 1)
        wait_back(1)
        return carry

    lax.fori_loop(0, (NCHUNK - 1) // 2, pair, 0)
    back(NCHUNK - 1, 0)
    wait_back(0)
    wait_idx(1)


@functools.partial(
    pl.kernel,
    out_type=jax.ShapeDtypeStruct((NC * NPAD, FM), jnp.float32),
    mesh=_mesh,
    scratch_types=[
        pltpu.VMEM((2, CH), jnp.int32),
        pltpu.VMEM((2, CH), jnp.int32),
        pltpu.VMEM((CH,), jnp.float32),
        pltpu.VMEM((CH,), jnp.float32),
        pltpu.VMEM((CH, FM), jnp.float32),
        pltpu.VMEM((CH, FM), jnp.float32),
        pltpu.VMEM((CH,), jnp.int32),
        pltpu.VMEM((CH,), jnp.int32),
        pltpu.VMEM_SHARED((NPAD, FM), jnp.float32),
        pltpu.SemaphoreType.DMA,
        pltpu.SemaphoreType.DMA,
        pltpu.SemaphoreType.DMA,
        pltpu.SemaphoreType.DMA,
        pltpu.SemaphoreType.DMA,
        pltpu.SemaphoreType.DMA,
    ],
)
def _sc_spmm(h_hbm, rc_hbm, nvh_hbm, zblk_hbm,
             out_hbm,
             i0, i1, n0, n1, buf0, buf1, si0, si1, acc,
             g0, g1, s0, s1, ii0, ii1):
    cid = lax.axis_index("c")
    sid = lax.axis_index("s")
    wid = sid * NC + cid
    pltpu.sync_copy(zblk_hbm, acc.at[pl.ds(sid * RPT, RPT)])
    plsc.subcore_barrier()

    def scale(buf, nbuf):
        def edge16(g, ecarry):
            vg = nbuf[pl.ds(g * 16, 16)]
            for jj in range(16):
                s = jnp.full((16,), vg[jj], jnp.float32)
                k = g * 16 + jj
                for v in range(FM // 16):
                    sl = pl.ds(v * 16, 16)
                    buf[k, sl] = buf[k, sl] * s
            return ecarry

        lax.fori_loop(0, CH // 16, edge16, 0)

    def fetch_idx(j, ibuf, nbuf, sem):
        pltpu.async_copy(rc_hbm.at[wid, j], ibuf, sem)
        pltpu.async_copy(nvh_hbm.at[wid, j], nbuf, sem)

    def wait_idx(ibuf, nbuf, sem):
        pltpu.make_async_copy(rc_hbm.at[0, 0], ibuf, sem).wait()
        pltpu.make_async_copy(nvh_hbm.at[0, 0], nbuf, sem).wait()

    def gather(ibuf, buf, sem):
        pltpu.async_copy(h_hbm.at[ibuf.at[1]], buf, sem)

    def wait_gather(buf, sem):
        pltpu.make_async_copy(h_hbm.at[i0.at[1]], buf, sem).wait()

    def sicopy(ibuf, sbuf):
        for v in range(CH // 16):
            sl = pl.ds(v * 16, 16)
            sbuf[sl] = ibuf[0, sl]

    def scatter(sbuf, buf, sem):
        pltpu.async_copy(buf, acc.at[sbuf], sem, add=True)

    def wait_scatter(buf, sem):
        pltpu.make_async_copy(buf, acc.at[si0], sem).wait()

    pltpu.sync_copy(rc_hbm.at[wid, 0], i0)
    pltpu.sync_copy(nvh_hbm.at[wid, 0], n0)
    gather(i0, buf0, g0)
    fetch_idx(1, i1, n1, ii1)

    def pair(t, carry):
        a = 2 * t
        wait_idx(i1, n1, ii1)
        gather(i1, buf1, g1)
        wait_gather(buf0, g0)
        scale(buf0, n0)
        sicopy(i0, si0)
        scatter(si0, buf0, s0)
        fetch_idx(a + 2, i0, n0, ii0)
        wait_gather(buf1, g1)
        scale(buf1, n1)
        sicopy(i1, si1)
        scatter(si1, buf1, s1)
        fetch_idx(a + 3, i1, n1, ii1)
        wait_scatter(buf0, s0)
        wait_idx(i0, n0, ii0)
        gather(i0, buf0, g0)
        wait_scatter(buf1, s1)
        return carry

    lax.fori_loop(0, (NCHUNK - 1) // 2, pair, 0)
    wait_gather(buf0, g0)
    scale(buf0, n0)
    sicopy(i0, si0)
    scatter(si0, buf0, s0)
    wait_scatter(buf0, s0)
    wait_idx(i1, n1, ii1)
    plsc.subcore_barrier()
    pltpu.sync_copy(acc.at[pl.ds(sid * RPT, RPT)],
                    out_hbm.at[pl.ds(cid * NPAD + sid * RPT, RPT)])


def _tc_dis_body(dp_ref, dis_ref, sl_ref):
    deg = dp_ref[0] + dp_ref[1] + (1.0 + 1e-6)
    dis = jnp.minimum(lax.rsqrt(deg), 1000.0)
    dis_ref[...] = dis
    sl_ref[...] = dis * dis


def _tc_dis(degp):
    return pl.pallas_call(
        _tc_dis_body,
        out_shape=[jax.ShapeDtypeStruct((8, NPAD // 8), jnp.float32),
                   jax.ShapeDtypeStruct((8, NPAD // 8), jnp.float32)],
    )(degp)


BR = 1280
GRID = NPAD // BR


def _tc_layer_body(p_ref, h_ref, sl_ref, wt_ref, b_ref, hout_ref, cs_ref):
    gid = pl.program_id(0)
    support = p_ref[0] + p_ref[1] + sl_ref[...] * h_ref[...]
    hh = jnp.dot(support, wt_ref[...], preferred_element_type=jnp.float32)
    hh = jnp.maximum(hh + b_ref[...], 0.0)
    rowid = gid * BR + lax.broadcasted_iota(jnp.int32, (BR, 1), 0)
    hh = jnp.where(rowid < NN, hh, 0.0)
    hout_ref[...] = hh

    @pl.when(gid == 0)
    def _():
        cs_ref[...] = jnp.zeros_like(cs_ref)

    cs_ref[...] += jnp.sum(hh, axis=0, keepdims=True)


def _tc_layer(partials, h, slv, wt, b):
    return pl.pallas_call(
        _tc_layer_body,
        grid=(GRID,),
        in_specs=[
            pl.BlockSpec((NC, BR, FM), lambda i: (0, i, 0)),
            pl.BlockSpec((BR, FM), lambda i: (i, 0)),
            pl.BlockSpec((BR, 1), lambda i: (i, 0)),
            pl.BlockSpec((FM, FM), lambda i: (0, 0)),
            pl.BlockSpec((1, FM), lambda i: (0, 0)),
        ],
        out_specs=[
            pl.BlockSpec((BR, FM), lambda i: (i, 0)),
            pl.BlockSpec((1, FM), lambda i: (0, 0)),
        ],
        out_shape=[jax.ShapeDtypeStruct((NPAD, FM), jnp.float32),
                   jax.ShapeDtypeStruct((1, FM), jnp.float32)],
    )(partials, h, slv, wt, b)


def _tc_attn_body(cs_ref, f1t_ref, f1b_ref, f2_ref, f2b_ref, cw_ref,
                  coef_ref):
    inv = 1.0 / (NN * FM)
    a1 = f1b_ref[...]
    for j in range(NL):
        ca_j = jnp.sum(cs_ref[j, :]) * inv
        a1 = a1 + ca_j * f1t_ref[j:j + 1, :]
    a1 = jnp.maximum(a1, 0.0)
    lane = lax.broadcasted_iota(jnp.int32, (1, FM), 1)
    out = jnp.zeros((1, FM), jnp.float32)
    for j in range(NL):
        oh = (lane == j).astype(jnp.float32)
        zj = jnp.sum(f2_ref[j:j + 1, :] * a1) + jnp.sum(f2b_ref[...] * oh)
        cj = jax.nn.sigmoid(zj) * jnp.sum(cw_ref[...] * oh)
        out = out + cj * oh
    coef_ref[...] = out


def _tc_attn(colsums, f1t, f1b, f2p, f2b, cwp):
    return pl.pallas_call(
        _tc_attn_body,
        out_shape=jax.ShapeDtypeStruct((1, FM), jnp.float32),
    )(colsums, f1t, f1b, f2p, f2b, cwp)


def _tc_combine_body(h0, h1, h2, h3, coef_ref, cb_ref, out_ref):
    lane = lax.broadcasted_iota(jnp.int32, (1, FM), 1)
    c = coef_ref[...]
    oh0 = (lane == 0).astype(jnp.float32)
    acc = jnp.sum(cb_ref[...] * oh0) + jnp.zeros_like(h0[...])
    hs = (h0, h1, h2, h3)
    for j in range(NL):
        cj = jnp.sum(c * (lane == j).astype(jnp.float32))
        acc = acc + cj * hs[j][...]
    out_ref[...] = acc


def _tc_combine(h0, h1, h2, h3, coef, cbp):
    blk = pl.BlockSpec((BR, FM), lambda i: (i, 0))
    one = pl.BlockSpec((1, FM), lambda i: (0, 0))
    return pl.pallas_call(
        _tc_combine_body,
        grid=(GRID,),
        in_specs=[blk, blk, blk, blk, one, one],
        out_specs=blk,
        out_shape=jax.ShapeDtypeStruct((NPAD, FM), jnp.float32),
    )(h0, h1, h2, h3, coef, cbp)


def kernel(x, edge_index, matrix, Ws, bs, fc1_w, fc1_b, fc2_w, fc2_b,
           cnn_w, cnn_b):
    rows = edge_index[0]
    cols = edge_index[1]
    matflat = matrix.reshape(-1)
    zrow = jnp.zeros((RPT,), jnp.float32)
    zblk = jnp.zeros((RPT, FM), jnp.float32)
    rc = jnp.stack([rows.reshape(NW, NCHUNK, CH),
                    cols.reshape(NW, NCHUNK, CH)], axis=2)
    rc = jnp.pad(rc, ((0, 0), (0, 1), (0, 0), (0, 0)))

    ev, degp = _sc_edge_vals(matflat, rc, zrow)
    dis8, sl8 = _tc_dis(degp.reshape(NC, 8, NPAD // 8))
    dis = dis8.reshape(NPAD)
    slv = sl8.reshape(NPAD, 1)
    nv = _sc_edge_norm(dis, rc, ev)

    h = jnp.pad(x, ((0, NPAD - NN), (0, 0)))
    nvh = jnp.pad(nv.reshape(NW, NCHUNK, CH), ((0, 0), (0, 1), (0, 0)))
    hs, css = [], []
    for l in range(NL):
        partials = _sc_spmm(h, rc, nvh, zblk).reshape(NC, NPAD, FM)
        h, cs = _tc_layer(partials, h, slv, Ws[l].T, bs[l].reshape(1, FM))
        hs.append(h)
        css.append(cs)

    colsums = jnp.concatenate(css, axis=0)
    k1 = fc1_w.shape[0]
    f1t = jnp.pad(fc1_w.T, ((0, 0), (0, FM - k1)))
    f1b = jnp.pad(fc1_b, (0, FM - k1)).reshape(1, FM)
    f2p = jnp.pad(fc2_w, ((0, 0), (0, FM - k1)))
    f2b = jnp.pad(fc2_b, (0, FM - NL)).reshape(1, FM)
    cwp = jnp.pad(cnn_w, (0, FM - NL)).reshape(1, FM)
    cbp = jnp.pad(cnn_b, (0, FM - 1)).reshape(1, FM)

    coef = _tc_attn(colsums, f1t, f1b, f2p, f2b, cwp)
    outp = _tc_combine(hs[0], hs[1], hs[2], hs[3], coef, cbp)
    return outp[:NN]


# speedup vs baseline: 1.4616x; 1.0674x over previous
import functools

import jax
import jax.numpy as jnp
from jax import lax
from jax.experimental import pallas as pl
from jax.experimental.pallas import tpu as pltpu
from jax.experimental.pallas import tpu_sc as plsc

NN = 10000
NPAD = 10240
EE = 320000
FM = 128
NL = 4
NC = 2
NS = 16
NW = NC * NS
EPW = EE // NW
CH = 80
NCHUNK = EPW // CH
RPT = NPAD // NS

_mesh = plsc.VectorSubcoreMesh(core_axis_name="c", subcore_axis_name="s")


@functools.partial(
    pl.kernel,
    out_type=(jax.ShapeDtypeStruct((EE,), jnp.float32),
              jax.ShapeDtypeStruct((NC * NPAD,), jnp.float32)),
    mesh=_mesh,
    scratch_types=[
        pltpu.VMEM((2, CH), jnp.int32),
        pltpu.VMEM((2, CH), jnp.int32),
        pltpu.VMEM((CH,), jnp.int32),
        pltpu.VMEM((CH,), jnp.int32),
        pltpu.VMEM((CH,), jnp.int32),
        pltpu.VMEM((CH,), jnp.int32),
        pltpu.VMEM((CH,), jnp.float32),
        pltpu.VMEM((CH,), jnp.float32),
        pltpu.VMEM_SHARED((NPAD,), jnp.float32),
        pltpu.SemaphoreType.DMA,
        pltpu.SemaphoreType.DMA,
        pltpu.SemaphoreType.DMA,
        pltpu.SemaphoreType.DMA,
        pltpu.SemaphoreType.DMA,
        pltpu.SemaphoreType.DMA,
        pltpu.SemaphoreType.DMA,
        pltpu.SemaphoreType.DMA,
    ],
)
def _sc_edge_vals(mat_hbm, rc_hbm, zrow_hbm,
                  ev_hbm, degp_hbm,
                  i0, i1, fi0, fi1, sr0, sr1, e0, e1, acc,
                  ii0, ii1, g0, g1, o0, o1, d0, d1):
    cid = lax.axis_index("c")
    sid = lax.axis_index("s")
    wid = sid * NC + cid
    pltpu.sync_copy(zrow_hbm, acc.at[pl.ds(sid * RPT, RPT)])
    plsc.subcore_barrier()

    ib = (i0, i1)
    fib = (fi0, fi1)
    srb = (sr0, sr1)
    eb = (e0, e1)
    gsem = (g0, g1)
    osem = (o0, o1)
    dsem = (d0, d1)

    def fetch_idx(j, b, sem):
        pltpu.async_copy(rc_hbm.at[wid, j], ib[b], sem)

    def wait_idx(b, sem):
        pltpu.make_async_copy(rc_hbm.at[0, 0], ib[b], sem).wait()

    def front(b):
        for v in range(CH // 16):
            sl = pl.ds(v * 16, 16)
            r = ib[b][0, sl]
            fib[b][sl] = r * NN + ib[b][1, sl]
            srb[b][sl] = r
        pltpu.async_copy(mat_hbm.at[fib[b]], eb[b], gsem[b])

    def back(j, b):
        pltpu.make_async_copy(mat_hbm.at[fi0], eb[b], gsem[b]).wait()
        base = wid * EPW + j * CH
        pltpu.async_copy(eb[b], ev_hbm.at[pl.ds(base, CH)], osem[b])
        pltpu.async_copy(eb[b], acc.at[srb[b]], dsem[b], add=True)

    def wait_back(b):
        pltpu.make_async_copy(eb[b], ev_hbm.at[pl.ds(0, CH)], osem[b]).wait()
        pltpu.make_async_copy(eb[b], acc.at[sr0], dsem[b]).wait()

    pltpu.sync_copy(rc_hbm.at[wid, 0], i0)
    front(0)
    fetch_idx(1, 1, ii1)

    def pair(t, carry):
        a = 2 * t
        wait_idx(1, ii1)
        front(1)
        fetch_idx(a + 2, 0, ii0)
        back(a, 0)
        wait_idx(0, ii0)
        wait_back(0)
        front(0)
        fetch_idx(a + 3, 1, ii1)
        back(a + 1, 1)
        wait_back(1)
        return carry

    lax.fori_loop(0, (NCHUNK - 1) // 2, pair, 0)
    back(NCHUNK - 1, 0)
    wait_back(0)
    wait_idx(1, ii1)
    plsc.subcore_barrier()
    pltpu.sync_copy(acc.at[pl.ds(sid * RPT, RPT)],
                    degp_hbm.at[pl.ds(cid * NPAD + sid * RPT, RPT)])


@functools.partial(
    pl.kernel,
    out_type=jax.ShapeDtypeStruct((NC * NPAD, FM), jnp.float32),
    mesh=_mesh,
    scratch_types=[
        pltpu.VMEM((2, CH), jnp.int32),
        pltpu.VMEM((2, CH), jnp.int32),
        pltpu.VMEM((CH,), jnp.float32),
        pltpu.VMEM((CH,), jnp.float32),
        pltpu.VMEM((CH, FM), jnp.float32),
        pltpu.VMEM((CH, FM), jnp.float32),
        pltpu.VMEM((CH,), jnp.int32),
        pltpu.VMEM((CH,), jnp.int32),
        pltpu.VMEM_SHARED((NPAD, FM), jnp.float32),
        pltpu.SemaphoreType.DMA,
        pltpu.SemaphoreType.DMA,
        pltpu.SemaphoreType.DMA,
        pltpu.SemaphoreType.DMA,
        pltpu.SemaphoreType.DMA,
        pltpu.SemaphoreType.DMA,
    ],
)
def _sc_spmm(h_hbm, rc_hbm, nvh_hbm, zblk_hbm,
             out_hbm,
             i0, i1, n0, n1, buf0, buf1, si0, si1, acc,
             g0, g1, s0, s1, ii0, ii1):
    cid = lax.axis_index("c")
    sid = lax.axis_index("s")
    wid = sid * NC + cid
    pltpu.sync_copy(zblk_hbm, acc.at[pl.ds(sid * RPT, RPT)])
    plsc.subcore_barrier()

    def scale(buf, nbuf):
        def edge16(g, ecarry):
            vg = nbuf[pl.ds(g * 16, 16)]
            for jj in range(16):
                s = jnp.full((16,), vg[jj], jnp.float32)
                k = g * 16 + jj
                for v in range(FM // 16):
                    sl = pl.ds(v * 16, 16)
                    buf[k, sl] = buf[k, sl] * s
            return ecarry

        lax.fori_loop(0, CH // 16, edge16, 0)

    def fetch_idx(j, ibuf, nbuf, sem):
        pltpu.async_copy(rc_hbm.at[wid, j], ibuf, sem)
        pltpu.async_copy(nvh_hbm.at[wid, j], nbuf, sem)

    def wait_idx(ibuf, nbuf, sem):
        pltpu.make_async_copy(rc_hbm.at[0, 0], ibuf, sem).wait()
        pltpu.make_async_copy(nvh_hbm.at[0, 0], nbuf, sem).wait()

    def gather(ibuf, buf, sem):
        pltpu.async_copy(h_hbm.at[ibuf.at[1]], buf, sem)

    def wait_gather(buf, sem):
        pltpu.make_async_copy(h_hbm.at[i0.at[1]], buf, sem).wait()

    def sicopy(ibuf, sbuf):
        for v in range(CH // 16):
            sl = pl.ds(v * 16, 16)
            sbuf[sl] = ibuf[0, sl]

    def scatter(sbuf, buf, sem):
        pltpu.async_copy(buf, acc.at[sbuf], sem, add=True)

    def wait_scatter(buf, sem):
        pltpu.make_async_copy(buf, acc.at[si0], sem).wait()

    pltpu.sync_copy(rc_hbm.at[wid, 0], i0)
    pltpu.sync_copy(nvh_hbm.at[wid, 0], n0)
    gather(i0, buf0, g0)
    fetch_idx(1, i1, n1, ii1)

    def pair(t, carry):
        a = 2 * t
        wait_idx(i1, n1, ii1)
        gather(i1, buf1, g1)
        wait_gather(buf0, g0)
        scale(buf0, n0)
        sicopy(i0, si0)
        scatter(si0, buf0, s0)
        fetch_idx(a + 2, i0, n0, ii0)
        wait_gather(buf1, g1)
        scale(buf1, n1)
        sicopy(i1, si1)
        scatter(si1, buf1, s1)
        fetch_idx(a + 3, i1, n1, ii1)
        wait_scatter(buf0, s0)
        wait_idx(i0, n0, ii0)
        gather(i0, buf0, g0)
        wait_scatter(buf1, s1)
        return carry

    lax.fori_loop(0, (NCHUNK - 1) // 2, pair, 0)
    wait_gather(buf0, g0)
    scale(buf0, n0)
    sicopy(i0, si0)
    scatter(si0, buf0, s0)
    wait_scatter(buf0, s0)
    wait_idx(i1, n1, ii1)
    plsc.subcore_barrier()
    pltpu.sync_copy(acc.at[pl.ds(sid * RPT, RPT)],
                    out_hbm.at[pl.ds(cid * NPAD + sid * RPT, RPT)])


@functools.partial(
    pl.kernel,
    out_type=(jax.ShapeDtypeStruct((NC * NPAD, FM), jnp.float32),
              jax.ShapeDtypeStruct((EE,), jnp.float32)),
    mesh=_mesh,
    scratch_types=[
        pltpu.VMEM((2, CH), jnp.int32),
        pltpu.VMEM((2, CH), jnp.int32),
        pltpu.VMEM((CH,), jnp.float32),
        pltpu.VMEM((CH,), jnp.float32),
        pltpu.VMEM((CH,), jnp.float32),
        pltpu.VMEM((CH,), jnp.float32),
        pltpu.VMEM((CH,), jnp.float32),
        pltpu.VMEM((CH,), jnp.float32),
        pltpu.VMEM((CH,), jnp.float32),
        pltpu.VMEM((CH,), jnp.float32),
        pltpu.VMEM((CH, FM), jnp.float32),
        pltpu.VMEM((CH, FM), jnp.float32),
        pltpu.VMEM((CH,), jnp.int32),
        pltpu.VMEM((CH,), jnp.int32),
        pltpu.VMEM_SHARED((NPAD, FM), jnp.float32),
        pltpu.SemaphoreType.DMA,
        pltpu.SemaphoreType.DMA,
        pltpu.SemaphoreType.DMA,
        pltpu.SemaphoreType.DMA,
        pltpu.SemaphoreType.DMA,
        pltpu.SemaphoreType.DMA,
        pltpu.SemaphoreType.DMA,
        pltpu.SemaphoreType.DMA,
        pltpu.SemaphoreType.DMA,
        pltpu.SemaphoreType.DMA,
    ],
)
def _sc_spmm0(h_hbm, rc_hbm, ev_hbm, dis_hbm, zblk_hbm,
              out_hbm, nv_hbm,
              i0, i1, e0, e1, dr0, dr1, dc0, dc1, n0, n1,
              buf0, buf1, si0, si1, acc,
              g0, g1, s0, s1, ii0, ii1, gd0, gd1, o0, o1):
    cid = lax.axis_index("c")
    sid = lax.axis_index("s")
    wid = sid * NC + cid
    pltpu.sync_copy(zblk_hbm, acc.at[pl.ds(sid * RPT, RPT)])
    plsc.subcore_barrier()

    ib = (i0, i1)
    eb = (e0, e1)
    drb = (dr0, dr1)
    dcb = (dc0, dc1)
    nb = (n0, n1)
    bufb = (buf0, buf1)
    sib = (si0, si1)
    gsem = (g0, g1)
    ssem = (s0, s1)
    isem = (ii0, ii1)
    dsem = (gd0, gd1)
    osem = (o0, o1)

    def fetch_idx(j, b):
        base = wid * EPW + j * CH
        pltpu.async_copy(rc_hbm.at[wid, j], ib[b], isem[b])
        pltpu.async_copy(ev_hbm.at[pl.ds(base, CH)], eb[b], isem[b])

    def wait_idx(b):
        pltpu.make_async_copy(rc_hbm.at[0, 0], ib[b], isem[b]).wait()
        pltpu.make_async_copy(ev_hbm.at[pl.ds(0, CH)], eb[b], isem[b]).wait()

    def gathers(b):
        pltpu.async_copy(h_hbm.at[ib[b].at[1]], bufb[b], gsem[b])
        pltpu.async_copy(dis_hbm.at[ib[b].at[0]], drb[b], dsem[b])
        pltpu.async_copy(dis_hbm.at[ib[b].at[1]], dcb[b], dsem[b])

    def wait_gather(b):
        pltpu.make_async_copy(h_hbm.at[i0.at[1]], bufb[b], gsem[b]).wait()
        pltpu.make_async_copy(dis_hbm.at[i0.at[0]], drb[b], dsem[b]).wait()
        pltpu.make_async_copy(dis_hbm.at[i0.at[0]], dcb[b], dsem[b]).wait()

    def nv_scale_scatter(j, b):
        for v in range(CH // 16):
            sl = pl.ds(v * 16, 16)
            nb[b][sl] = drb[b][sl] * eb[b][sl] * dcb[b][sl]
            sib[b][sl] = ib[b][0, sl]
        base = wid * EPW + j * CH
        pltpu.async_copy(nb[b], nv_hbm.at[pl.ds(base, CH)], osem[b])

        def edge16(g, ecarry):
            vg = nb[b][pl.ds(g * 16, 16)]
            for jj in range(16):
                s = jnp.full((16,), vg[jj], jnp.float32)
                k = g * 16 + jj
                for v2 in range(FM // 16):
                    sl2 = pl.ds(v2 * 16, 16)
                    bufb[b][k, sl2] = bufb[b][k, sl2] * s
            return ecarry

        lax.fori_loop(0, CH // 16, edge16, 0)
        pltpu.make_async_copy(nb[b], nv_hbm.at[pl.ds(0, CH)], osem[b]).wait()
        pltpu.async_copy(bufb[b], acc.at[sib[b]], ssem[b], add=True)

    def wait_scatter(b):
        pltpu.make_async_copy(bufb[b], acc.at[si0], ssem[b]).wait()

    pltpu.sync_copy(rc_hbm.at[wid, 0], i0)
    base0 = wid * EPW
    pltpu.sync_copy(ev_hbm.at[pl.ds(base0, CH)], e0)
    gathers(0)
    fetch_idx(1, 1)

    def pair(t, carry):
        a = 2 * t
        wait_idx(1)
        gathers(1)
        wait_gather(0)
        nv_scale_scatter(a, 0)
        fetch_idx(a + 2, 0)
        wait_gather(1)
        nv_scale_scatter(a + 1, 1)
        fetch_idx(a + 3, 1)
        wait_scatter(0)
        wait_idx(0)
        gathers(0)
        wait_scatter(1)
        return carry

    lax.fori_loop(0, (NCHUNK - 1) // 2, pair, 0)
    wait_gather(0)
    nv_scale_scatter(NCHUNK - 1, 0)
    wait_scatter(0)
    wait_idx(1)
    plsc.subcore_barrier()
    pltpu.sync_copy(acc.at[pl.ds(sid * RPT, RPT)],
                    out_hbm.at[pl.ds(cid * NPAD + sid * RPT, RPT)])


def _tc_dis_body(dp_ref, dis_ref, sl_ref):
    deg = dp_ref[0] + dp_ref[1] + (1.0 + 1e-6)
    dis = jnp.minimum(lax.rsqrt(deg), 1000.0)
    dis_ref[...] = dis
    sl_ref[...] = dis * dis


def _tc_dis(degp):
    return pl.pallas_call(
        _tc_dis_body,
        out_shape=[jax.ShapeDtypeStruct((8, NPAD // 8), jnp.float32),
                   jax.ShapeDtypeStruct((8, NPAD // 8), jnp.float32)],
    )(degp)


BR = 1280
GRID = NPAD // BR


def _tc_layer_body(p_ref, h_ref, sl_ref, wt_ref, b_ref, hout_ref, cs_ref):
    gid = pl.program_id(0)
    support = p_ref[0] + p_ref[1] + sl_ref[...] * h_ref[...]
    hh = jnp.dot(support, wt_ref[...], preferred_element_type=jnp.float32)
    hh = jnp.maximum(hh + b_ref[...], 0.0)
    rowid = gid * BR + lax.broadcasted_iota(jnp.int32, (BR, 1), 0)
    hh = jnp.where(rowid < NN, hh, 0.0)
    hout_ref[...] = hh

    @pl.when(gid == 0)
    def _():
        cs_ref[...] = jnp.zeros_like(cs_ref)

    cs_ref[...] += jnp.sum(hh, axis=0, keepdims=True)


def _tc_layer(partials, h, slv, wt, b):
    return pl.pallas_call(
        _tc_layer_body,
        grid=(GRID,),
        in_specs=[
            pl.BlockSpec((NC, BR, FM), lambda i: (0, i, 0)),
            pl.BlockSpec((BR, FM), lambda i: (i, 0)),
            pl.BlockSpec((BR, 1), lambda i: (i, 0)),
            pl.BlockSpec((FM, FM), lambda i: (0, 0)),
            pl.BlockSpec((1, FM), lambda i: (0, 0)),
        ],
        out_specs=[
            pl.BlockSpec((BR, FM), lambda i: (i, 0)),
            pl.BlockSpec((1, FM), lambda i: (0, 0)),
        ],
        out_shape=[jax.ShapeDtypeStruct((NPAD, FM), jnp.float32),
                   jax.ShapeDtypeStruct((1, FM), jnp.float32)],
    )(partials, h, slv, wt, b)


def _tc_attn_body(cs_ref, f1t_ref, f1b_ref, f2_ref, f2b_ref, cw_ref,
                  coef_ref):
    inv = 1.0 / (NN * FM)
    a1 = f1b_ref[...]
    for j in range(NL):
        ca_j = jnp.sum(cs_ref[j, :]) * inv
        a1 = a1 + ca_j * f1t_ref[j:j + 1, :]
    a1 = jnp.maximum(a1, 0.0)
    lane = lax.broadcasted_iota(jnp.int32, (1, FM), 1)
    out = jnp.zeros((1, FM), jnp.float32)
    for j in range(NL):
        oh = (lane == j).astype(jnp.float32)
        zj = jnp.sum(f2_ref[j:j + 1, :] * a1) + jnp.sum(f2b_ref[...] * oh)
        cj = jax.nn.sigmoid(zj) * jnp.sum(cw_ref[...] * oh)
        out = out + cj * oh
    coef_ref[...] = out


def _tc_attn(colsums, f1t, f1b, f2p, f2b, cwp):
    return pl.pallas_call(
        _tc_attn_body,
        out_shape=jax.ShapeDtypeStruct((1, FM), jnp.float32),
    )(colsums, f1t, f1b, f2p, f2b, cwp)


def _tc_combine_body(h0, h1, h2, h3, coef_ref, cb_ref, out_ref):
    lane = lax.broadcasted_iota(jnp.int32, (1, FM), 1)
    c = coef_ref[...]
    oh0 = (lane == 0).astype(jnp.float32)
    acc = jnp.sum(cb_ref[...] * oh0) + jnp.zeros_like(h0[...])
    hs = (h0, h1, h2, h3)
    for j in range(NL):
        cj = jnp.sum(c * (lane == j).astype(jnp.float32))
        acc = acc + cj * hs[j][...]
    out_ref[...] = acc


def _tc_combine(h0, h1, h2, h3, coef, cbp):
    blk = pl.BlockSpec((BR, FM), lambda i: (i, 0))
    one = pl.BlockSpec((1, FM), lambda i: (0, 0))
    return pl.pallas_call(
        _tc_combine_body,
        grid=(GRID,),
        in_specs=[blk, blk, blk, blk, one, one],
        out_specs=blk,
        out_shape=jax.ShapeDtypeStruct((NPAD, FM), jnp.float32),
    )(h0, h1, h2, h3, coef, cbp)


def kernel(x, edge_index, matrix, Ws, bs, fc1_w, fc1_b, fc2_w, fc2_b,
           cnn_w, cnn_b):
    rows = edge_index[0]
    cols = edge_index[1]
    matflat = matrix.reshape(-1)
    zrow = jnp.zeros((RPT,), jnp.float32)
    zblk = jnp.zeros((RPT, FM), jnp.float32)
    rc = jnp.stack([rows.reshape(NW, NCHUNK, CH),
                    cols.reshape(NW, NCHUNK, CH)], axis=2)
    rc = jnp.pad(rc, ((0, 0), (0, 1), (0, 0), (0, 0)))

    ev, degp = _sc_edge_vals(matflat, rc, zrow)
    dis8, sl8 = _tc_dis(degp.reshape(NC, 8, NPAD // 8))
    dis = dis8.reshape(NPAD)
    slv = sl8.reshape(NPAD, 1)
    h = jnp.pad(x, ((0, NPAD - NN), (0, 0)))
    hs, css = [], []
    p0, nv = _sc_spmm0(h, rc, ev, dis, zblk)
    nvh = jnp.pad(nv.reshape(NW, NCHUNK, CH), ((0, 0), (0, 1), (0, 0)))
    h, cs = _tc_layer(p0.reshape(NC, NPAD, FM), h, slv, Ws[0].T,
                      bs[0].reshape(1, FM))
    hs.append(h)
    css.append(cs)
    for l in range(1, NL):
        partials = _sc_spmm(h, rc, nvh, zblk).reshape(NC, NPAD, FM)
        h, cs = _tc_layer(partials, h, slv, Ws[l].T, bs[l].reshape(1, FM))
        hs.append(h)
        css.append(cs)

    colsums = jnp.concatenate(css, axis=0)
    k1 = fc1_w.shape[0]
    f1t = jnp.pad(fc1_w.T, ((0, 0), (0, FM - k1)))
    f1b = jnp.pad(fc1_b, (0, FM - k1)).reshape(1, FM)
    f2p = jnp.pad(fc2_w, ((0, 0), (0, FM - k1)))
    f2b = jnp.pad(fc2_b, (0, FM - NL)).reshape(1, FM)
    cwp = jnp.pad(cnn_w, (0, FM - NL)).reshape(1, FM)
    cbp = jnp.pad(cnn_b, (0, FM - 1)).reshape(1, FM)

    coef = _tc_attn(colsums, f1t, f1b, f2p, f2b, cwp)
    outp = _tc_combine(hs[0], hs[1], hs[2], hs[3], coef, cbp)
    return outp[:NN]
